# baseline (speedup 1.0000x reference)
import functools

import jax
import jax.numpy as jnp
from jax import lax
from jax.experimental import pallas as pl
from jax.experimental.pallas import tpu as pltpu
from jax.experimental.pallas import tpu_sc as plsc

VOCAB = 1000000
EMBED = 32
NCLS = 2
BATCH = 4096
SEQ = 200

NC = 2
NS = 16
L = 16
NW = NC * NS
BPW = BATCH // NW
CH = 4
NCHUNK = BPW // CH
SP = 208
NT = SP // L
SPLITS = ((0, 128), (128, 80))

BKV = 8000
PBLK = 8192
NBLK = VOCAB // BKV
PAD_ID = BKV

_mesh = plsc.VectorSubcoreMesh(core_axis_name="c", subcore_axis_name="s")


def _bf16_hi(c):
    u = lax.bitcast_convert_type(c, jnp.uint32)
    return (u + jnp.uint32(0x7FFF) + ((u >> 16) & jnp.uint32(1))) >> 16


def _proj_body(t_ref, w_ref, b_ref, p_ref):
    w = w_ref[...] * (1.0 / SEQ)
    bb = b_ref[...] * (1.0 / SEQ)
    out2 = lax.dot_general(w, t_ref[...],
                           dimension_numbers=(((0,), (1,)), ((), ())),
                           preferred_element_type=jnp.float32)
    c0 = out2[0] + bb[0, 0]
    c1 = out2[1] + bb[0, 1]
    u = _bf16_hi(c0) | (_bf16_hi(c1) << 16)
    p_ref[pl.ds(0, BKV)] = lax.bitcast_convert_type(u, jnp.float32)
    p_ref[pl.ds(BKV, PBLK - BKV)] = jnp.zeros((PBLK - BKV,), jnp.float32)


_proj = pl.pallas_call(
    _proj_body,
    grid=(NBLK,),
    in_specs=[
        pl.BlockSpec((BKV, EMBED), lambda i: (i, 0)),
        pl.BlockSpec((EMBED, NCLS), lambda i: (0, 0)),
        pl.BlockSpec((1, NCLS), lambda i: (0, 0)),
    ],
    out_specs=pl.BlockSpec((PBLK,), lambda i: (i,)),
    out_shape=jax.ShapeDtypeStruct((NBLK * PBLK,), jnp.float32),
)


@functools.partial(
    pl.kernel,
    mesh=_mesh,
    compiler_params=pltpu.CompilerParams(use_tc_tiling_on_sc=False),
    out_type=jax.ShapeDtypeStruct((BATCH, EMBED), jnp.float32),
    scratch_types=[
        pltpu.VMEM((CH, SP), jnp.int32),
        pltpu.VMEM((CH, SP), jnp.float32),
        pltpu.VMEM((BPW, EMBED), jnp.float32),
        pltpu.SemaphoreType.DMA,
    ],
)
def _pool_gather(idx_hbm, pf_hbm, out_hbm, idx_v, rows_v, acc_v, sem):
    wid = lax.axis_index("s") * NC + lax.axis_index("c")
    base = wid * BPW

    def chunk_body(ci, carry):
        row0 = base + ci * CH
        pltpu.sync_copy(idx_hbm.at[pl.ds(row0, CH)], idx_v)
        cps = []
        for r in range(CH):
            for off, n in SPLITS:
                cps.append(pltpu.async_copy(
                    pf_hbm.at[idx_v.at[r, pl.ds(off, n)]],
                    rows_v.at[r, pl.ds(off, n)], sem))
        for cp in cps:
            cp.wait()
        for r in range(CH):
            a0 = jnp.zeros((L,), jnp.float32)
            a1 = jnp.zeros((L,), jnp.float32)
            for t in range(NT):
                v = rows_v[r, pl.ds(t * L, L)]
                ui = lax.bitcast_convert_type(v, jnp.uint32)
                e0 = lax.bitcast_convert_type(ui << 16, jnp.float32)
                e1 = lax.bitcast_convert_type(
                    ui & jnp.uint32(0xFFFF0000), jnp.float32)
                a0 = a0 + e0
                a1 = a1 + e1
            acc_v[ci * CH + r, pl.ds(0, L)] = a0
            acc_v[ci * CH + r, pl.ds(L, L)] = a1
        return carry

    lax.fori_loop(0, NCHUNK, chunk_body, 0)
    pltpu.sync_copy(acc_v, out_hbm.at[pl.ds(base, BPW)])


def _head_body(p_ref, m_ref, o_ref):
    o_ref[...] = jnp.dot(p_ref[...], m_ref[...],
                         preferred_element_type=jnp.float32,
                         precision=lax.Precision.HIGHEST)


_head = pl.pallas_call(
    _head_body,
    out_shape=jax.ShapeDtypeStruct((BATCH, NCLS), jnp.float32),
)


def kernel(x, table, W, b):
    xi = x.astype(jnp.int32)
    ids = PBLK * (xi // BKV) + xi % BKV
    idx2 = jnp.concatenate(
        [ids, jnp.full((BATCH, SP - SEQ), PAD_ID, jnp.int32)], axis=1)
    p = _proj(table, W, b.reshape(1, NCLS))
    staged = _pool_gather(idx2, p)
    m = jnp.repeat(jnp.eye(NCLS, dtype=jnp.float32), L, axis=0)
    return _head(staged, m)


# speedup vs baseline: 1.0023x; 1.0023x over previous
import functools

import jax
import jax.numpy as jnp
from jax import lax
from jax.experimental import pallas as pl
from jax.experimental.pallas import tpu as pltpu
from jax.experimental.pallas import tpu_sc as plsc

VOCAB = 1000000
EMBED = 32
NCLS = 2
BATCH = 4096
SEQ = 200

NC = 2
NS = 16
L = 16
NW = NC * NS
BPW = BATCH // NW
CH = 4
NCHUNK = BPW // CH
SP = 208
NT = SP // L
SPLITS = ((0, 128), (128, 80))

BKV = 8000
PBLK = 8192
NBLK = VOCAB // BKV
PAD_ID = BKV

_mesh = plsc.VectorSubcoreMesh(core_axis_name="c", subcore_axis_name="s")


def _bf16_hi(c):
    u = lax.bitcast_convert_type(c, jnp.uint32)
    return (u + jnp.uint32(0x7FFF) + ((u >> 16) & jnp.uint32(1))) >> 16


def _proj_body(t_ref, w_ref, b_ref, p_ref):
    w = w_ref[...] * (1.0 / SEQ)
    bb = b_ref[...] * (1.0 / SEQ)
    out2 = lax.dot_general(w, t_ref[...],
                           dimension_numbers=(((0,), (1,)), ((), ())),
                           preferred_element_type=jnp.float32)
    c0 = out2[0] + bb[0, 0]
    c1 = out2[1] + bb[0, 1]
    u = _bf16_hi(c0) | (_bf16_hi(c1) << 16)
    p_ref[pl.ds(0, BKV)] = lax.bitcast_convert_type(u, jnp.float32)
    p_ref[pl.ds(BKV, PBLK - BKV)] = jnp.zeros((PBLK - BKV,), jnp.float32)


_proj = pl.pallas_call(
    _proj_body,
    grid=(NBLK,),
    in_specs=[
        pl.BlockSpec((BKV, EMBED), lambda i: (i, 0)),
        pl.BlockSpec((EMBED, NCLS), lambda i: (0, 0)),
        pl.BlockSpec((1, NCLS), lambda i: (0, 0)),
    ],
    out_specs=pl.BlockSpec((PBLK,), lambda i: (i,)),
    out_shape=jax.ShapeDtypeStruct((NBLK * PBLK,), jnp.float32),
)


@functools.partial(
    pl.kernel,
    mesh=_mesh,
    compiler_params=pltpu.CompilerParams(use_tc_tiling_on_sc=False),
    out_type=jax.ShapeDtypeStruct((BATCH, EMBED), jnp.float32),
    scratch_types=[
        pltpu.VMEM((CH, SP), jnp.int32),
        pltpu.VMEM((CH, SP), jnp.int32),
        pltpu.VMEM((CH, SP), jnp.float32),
        pltpu.VMEM((CH, SP), jnp.float32),
        pltpu.VMEM((BPW, EMBED), jnp.float32),
        pltpu.SemaphoreType.DMA,
        pltpu.SemaphoreType.DMA,
    ],
)
def _pool_gather(idx_hbm, pf_hbm, out_hbm,
                 idx_a, idx_b, rows_a, rows_b, acc_v, sem_a, sem_b):
    wid = lax.axis_index("s") * NC + lax.axis_index("c")
    base = wid * BPW

    def stage(ci, idx_v, rows_v, sem):
        pltpu.sync_copy(idx_hbm.at[pl.ds(base + ci * CH, CH)], idx_v)
        for r in range(CH):
            for off, n in SPLITS:
                pltpu.async_copy(
                    pf_hbm.at[idx_v.at[r, pl.ds(off, n)]],
                    rows_v.at[r, pl.ds(off, n)], sem)

    def drain(rows_v, sem):
        for r in range(CH):
            for off, n in SPLITS:
                pltpu.make_async_copy(
                    pf_hbm.at[pl.ds(0, n)],
                    rows_v.at[r, pl.ds(off, n)], sem).wait()

    def consume(ci, rows_v):
        for r in range(CH):
            a0 = jnp.zeros((L,), jnp.float32)
            a1 = jnp.zeros((L,), jnp.float32)
            for t in range(NT):
                v = rows_v[r, pl.ds(t * L, L)]
                ui = lax.bitcast_convert_type(v, jnp.uint32)
                a0 = a0 + lax.bitcast_convert_type(ui << 16, jnp.float32)
                a1 = a1 + lax.bitcast_convert_type(
                    ui & jnp.uint32(0xFFFF0000), jnp.float32)
            acc_v[ci * CH + r, pl.ds(0, L)] = a0
            acc_v[ci * CH + r, pl.ds(L, L)] = a1

    stage(0, idx_a, rows_a, sem_a)

    def pair_body(j, carry):
        stage(2 * j + 1, idx_b, rows_b, sem_b)
        drain(rows_a, sem_a)
        consume(2 * j, rows_a)

        @pl.when(j + 1 < NCHUNK // 2)
        def _():
            stage(2 * j + 2, idx_a, rows_a, sem_a)

        drain(rows_b, sem_b)
        consume(2 * j + 1, rows_b)
        return carry

    lax.fori_loop(0, NCHUNK // 2, pair_body, 0)
    pltpu.sync_copy(acc_v, out_hbm.at[pl.ds(base, BPW)])


def _head_body(p_ref, m_ref, o_ref):
    o_ref[...] = jnp.dot(p_ref[...], m_ref[...],
                         preferred_element_type=jnp.float32,
                         precision=lax.Precision.HIGHEST)


_head = pl.pallas_call(
    _head_body,
    out_shape=jax.ShapeDtypeStruct((BATCH, NCLS), jnp.float32),
)


def kernel(x, table, W, b):
    xi = x.astype(jnp.int32)
    ids = PBLK * (xi // BKV) + xi % BKV
    idx2 = jnp.concatenate(
        [ids, jnp.full((BATCH, SP - SEQ), PAD_ID, jnp.int32)], axis=1)
    p = _proj(table, W, b.reshape(1, NCLS))
    staged = _pool_gather(idx2, p)
    m = jnp.repeat(jnp.eye(NCLS, dtype=jnp.float32), L, axis=0)
    return _head(staged, m)


# speedup vs baseline: 1.0903x; 1.0878x over previous
import functools

import jax
import jax.numpy as jnp
from jax import lax
from jax.experimental import pallas as pl
from jax.experimental.pallas import tpu as pltpu
from jax.experimental.pallas import tpu_sc as plsc

VOCAB = 1000000
EMBED = 32
NCLS = 2
BATCH = 4096
SEQ = 200

NC = 2
NS = 16
L = 16
NW = NC * NS
BPW = BATCH // NW
CH = 8
NCHUNK = BPW // CH
SP = 208
NT = SP // L
SPLITS = ((0, 128), (128, 80))

BKV = 25000
PBLK = 32768
NBLK = VOCAB // BKV
PAD_ID = BKV

_mesh = plsc.VectorSubcoreMesh(core_axis_name="c", subcore_axis_name="s")


def _bf16_hi(c):
    u = lax.bitcast_convert_type(c, jnp.uint32)
    return (u + jnp.uint32(0x7FFF) + ((u >> 16) & jnp.uint32(1))) >> 16


def _proj_body(t_ref, w_ref, b_ref, p_ref):
    w = w_ref[...] * (1.0 / SEQ)
    bb = b_ref[...] * (1.0 / SEQ)
    out2 = lax.dot_general(w, t_ref[...],
                           dimension_numbers=(((0,), (1,)), ((), ())),
                           preferred_element_type=jnp.float32)
    c0 = out2[0] + bb[0, 0]
    c1 = out2[1] + bb[0, 1]
    u = _bf16_hi(c0) | (_bf16_hi(c1) << 16)
    p_ref[pl.ds(0, BKV)] = lax.bitcast_convert_type(u, jnp.float32)
    p_ref[pl.ds(BKV, PBLK - BKV)] = jnp.zeros((PBLK - BKV,), jnp.float32)


_proj = pl.pallas_call(
    _proj_body,
    grid=(NBLK,),
    in_specs=[
        pl.BlockSpec((BKV, EMBED), lambda i: (i, 0)),
        pl.BlockSpec((EMBED, NCLS), lambda i: (0, 0)),
        pl.BlockSpec((1, NCLS), lambda i: (0, 0)),
    ],
    out_specs=pl.BlockSpec((PBLK,), lambda i: (i,)),
    out_shape=jax.ShapeDtypeStruct((NBLK * PBLK,), jnp.float32),
)


@functools.partial(
    pl.kernel,
    mesh=_mesh,
    compiler_params=pltpu.CompilerParams(use_tc_tiling_on_sc=False),
    out_type=jax.ShapeDtypeStruct((BATCH, EMBED), jnp.float32),
    scratch_types=[
        pltpu.VMEM((CH, SP), jnp.int32),
        pltpu.VMEM((CH, SP), jnp.int32),
        pltpu.VMEM((CH, SP), jnp.float32),
        pltpu.VMEM((CH, SP), jnp.float32),
        pltpu.VMEM((BPW, EMBED), jnp.float32),
        pltpu.SemaphoreType.DMA,
        pltpu.SemaphoreType.DMA,
    ],
)
def _pool_gather(idx_hbm, pf_hbm, out_hbm,
                 idx_a, idx_b, rows_a, rows_b, acc_v, sem_a, sem_b):
    wid = lax.axis_index("s") * NC + lax.axis_index("c")
    base = wid * BPW

    def stage(ci, idx_v, rows_v, sem):
        pltpu.sync_copy(idx_hbm.at[pl.ds(base + ci * CH, CH)], idx_v)
        for r in range(CH):
            for off, n in SPLITS:
                pltpu.async_copy(
                    pf_hbm.at[idx_v.at[r, pl.ds(off, n)]],
                    rows_v.at[r, pl.ds(off, n)], sem)

    def drain(rows_v, sem):
        for r in range(CH):
            for off, n in SPLITS:
                pltpu.make_async_copy(
                    pf_hbm.at[pl.ds(0, n)],
                    rows_v.at[r, pl.ds(off, n)], sem).wait()

    def consume(ci, rows_v):
        for r in range(CH):
            a0 = jnp.zeros((L,), jnp.float32)
            a1 = jnp.zeros((L,), jnp.float32)
            for t in range(NT):
                v = rows_v[r, pl.ds(t * L, L)]
                ui = lax.bitcast_convert_type(v, jnp.uint32)
                a0 = a0 + lax.bitcast_convert_type(ui << 16, jnp.float32)
                a1 = a1 + lax.bitcast_convert_type(
                    ui & jnp.uint32(0xFFFF0000), jnp.float32)
            acc_v[ci * CH + r, pl.ds(0, L)] = a0
            acc_v[ci * CH + r, pl.ds(L, L)] = a1

    stage(0, idx_a, rows_a, sem_a)

    def pair_body(j, carry):
        stage(2 * j + 1, idx_b, rows_b, sem_b)
        drain(rows_a, sem_a)
        consume(2 * j, rows_a)

        @pl.when(j + 1 < NCHUNK // 2)
        def _():
            stage(2 * j + 2, idx_a, rows_a, sem_a)

        drain(rows_b, sem_b)
        consume(2 * j + 1, rows_b)
        return carry

    lax.fori_loop(0, NCHUNK // 2, pair_body, 0)
    pltpu.sync_copy(acc_v, out_hbm.at[pl.ds(base, BPW)])


def _head_body(p_ref, m_ref, o_ref):
    o_ref[...] = jnp.dot(p_ref[...], m_ref[...],
                         preferred_element_type=jnp.float32,
                         precision=lax.Precision.HIGHEST)


_head = pl.pallas_call(
    _head_body,
    out_shape=jax.ShapeDtypeStruct((BATCH, NCLS), jnp.float32),
)


def kernel(x, table, W, b):
    xi = x.astype(jnp.int32)
    ids = PBLK * (xi // BKV) + xi % BKV
    idx2 = jnp.concatenate(
        [ids, jnp.full((BATCH, SP - SEQ), PAD_ID, jnp.int32)], axis=1)
    p = _proj(table, W, b.reshape(1, NCLS))
    staged = _pool_gather(idx2, p)
    m = jnp.repeat(jnp.eye(NCLS, dtype=jnp.float32), L, axis=0)
    return _head(staged, m)


# speedup vs baseline: 1.0910x; 1.0006x over previous
import functools

import jax
import jax.numpy as jnp
from jax import lax
from jax.experimental import pallas as pl
from jax.experimental.pallas import tpu as pltpu
from jax.experimental.pallas import tpu_sc as plsc

VOCAB = 1000000
EMBED = 32
NCLS = 2
BATCH = 4096
SEQ = 200

NC = 2
NS = 16
L = 16
NW = NC * NS
BPW = BATCH // NW
CH = 16
NCHUNK = BPW // CH
SP = 208
NT = SP // L
SPLITS = ((0, 128), (128, 80))

BKV = 25000
PBLK = 32768
NBLK = VOCAB // BKV
PAD_ID = BKV

_mesh = plsc.VectorSubcoreMesh(core_axis_name="c", subcore_axis_name="s")


def _bf16_hi(c):
    u = lax.bitcast_convert_type(c, jnp.uint32)
    return (u + jnp.uint32(0x7FFF) + ((u >> 16) & jnp.uint32(1))) >> 16


def _proj_body(t_ref, w_ref, b_ref, p_ref):
    w = w_ref[...] * (1.0 / SEQ)
    bb = b_ref[...] * (1.0 / SEQ)
    out2 = lax.dot_general(w, t_ref[...],
                           dimension_numbers=(((0,), (1,)), ((), ())),
                           preferred_element_type=jnp.float32)
    c0 = out2[0] + bb[0, 0]
    c1 = out2[1] + bb[0, 1]
    u = _bf16_hi(c0) | (_bf16_hi(c1) << 16)
    p_ref[pl.ds(0, BKV)] = lax.bitcast_convert_type(u, jnp.float32)
    p_ref[pl.ds(BKV, PBLK - BKV)] = jnp.zeros((PBLK - BKV,), jnp.float32)


_proj = pl.pallas_call(
    _proj_body,
    grid=(NBLK,),
    in_specs=[
        pl.BlockSpec((BKV, EMBED), lambda i: (i, 0)),
        pl.BlockSpec((EMBED, NCLS), lambda i: (0, 0)),
        pl.BlockSpec((1, NCLS), lambda i: (0, 0)),
    ],
    out_specs=pl.BlockSpec((PBLK,), lambda i: (i,)),
    out_shape=jax.ShapeDtypeStruct((NBLK * PBLK,), jnp.float32),
)


@functools.partial(
    pl.kernel,
    mesh=_mesh,
    compiler_params=pltpu.CompilerParams(use_tc_tiling_on_sc=False),
    out_type=jax.ShapeDtypeStruct((BATCH, EMBED), jnp.float32),
    scratch_types=[
        pltpu.VMEM((CH, SP), jnp.int32),
        pltpu.VMEM((CH, SP), jnp.int32),
        pltpu.VMEM((CH, SP), jnp.float32),
        pltpu.VMEM((CH, SP), jnp.float32),
        pltpu.VMEM((BPW, EMBED), jnp.float32),
        pltpu.SemaphoreType.DMA,
        pltpu.SemaphoreType.DMA,
    ],
)
def _pool_gather(idx_hbm, pf_hbm, out_hbm,
                 idx_a, idx_b, rows_a, rows_b, acc_v, sem_a, sem_b):
    wid = lax.axis_index("s") * NC + lax.axis_index("c")
    base = wid * BPW

    def stage(ci, idx_v, rows_v, sem):
        pltpu.sync_copy(idx_hbm.at[pl.ds(base + ci * CH, CH)], idx_v)
        for r in range(CH):
            for off, n in SPLITS:
                pltpu.async_copy(
                    pf_hbm.at[idx_v.at[r, pl.ds(off, n)]],
                    rows_v.at[r, pl.ds(off, n)], sem)

    def drain(rows_v, sem):
        for r in range(CH):
            for off, n in SPLITS:
                pltpu.make_async_copy(
                    pf_hbm.at[pl.ds(0, n)],
                    rows_v.at[r, pl.ds(off, n)], sem).wait()

    def consume(ci, rows_v):
        for r in range(CH):
            a0 = jnp.zeros((L,), jnp.float32)
            a1 = jnp.zeros((L,), jnp.float32)
            for t in range(NT):
                v = rows_v[r, pl.ds(t * L, L)]
                ui = lax.bitcast_convert_type(v, jnp.uint32)
                a0 = a0 + lax.bitcast_convert_type(ui << 16, jnp.float32)
                a1 = a1 + lax.bitcast_convert_type(
                    ui & jnp.uint32(0xFFFF0000), jnp.float32)
            acc_v[ci * CH + r, pl.ds(0, L)] = a0
            acc_v[ci * CH + r, pl.ds(L, L)] = a1

    stage(0, idx_a, rows_a, sem_a)

    def pair_body(j, carry):
        stage(2 * j + 1, idx_b, rows_b, sem_b)
        drain(rows_a, sem_a)
        consume(2 * j, rows_a)

        @pl.when(j + 1 < NCHUNK // 2)
        def _():
            stage(2 * j + 2, idx_a, rows_a, sem_a)

        drain(rows_b, sem_b)
        consume(2 * j + 1, rows_b)
        return carry

    lax.fori_loop(0, NCHUNK // 2, pair_body, 0)
    pltpu.sync_copy(acc_v, out_hbm.at[pl.ds(base, BPW)])


def _head_body(p_ref, m_ref, o_ref):
    o_ref[...] = jnp.dot(p_ref[...], m_ref[...],
                         preferred_element_type=jnp.float32,
                         precision=lax.Precision.HIGHEST)


_head = pl.pallas_call(
    _head_body,
    out_shape=jax.ShapeDtypeStruct((BATCH, NCLS), jnp.float32),
)


def kernel(x, table, W, b):
    xi = x.astype(jnp.int32)
    ids = PBLK * (xi // BKV) + xi % BKV
    idx2 = jnp.concatenate(
        [ids, jnp.full((BATCH, SP - SEQ), PAD_ID, jnp.int32)], axis=1)
    p = _proj(table, W, b.reshape(1, NCLS))
    staged = _pool_gather(idx2, p)
    m = jnp.repeat(jnp.eye(NCLS, dtype=jnp.float32), L, axis=0)
    return _head(staged, m)


# speedup vs baseline: 1.2216x; 1.1197x over previous
import functools

import jax
import jax.numpy as jnp
from jax import lax
from jax.experimental import pallas as pl
from jax.experimental.pallas import tpu as pltpu
from jax.experimental.pallas import tpu_sc as plsc

VOCAB = 1000000
EMBED = 32
NCLS = 2
BATCH = 4096
SEQ = 200

NC = 2
NS = 16
L = 16
NW = NC * NS
BPW = BATCH // NW
CH = 4
NCHUNK = BPW // CH
HALF = SEQ // 2
UNROLL = 4

_mesh = plsc.VectorSubcoreMesh(core_axis_name="c", subcore_axis_name="s")


@functools.partial(
    pl.kernel,
    mesh=_mesh,
    compiler_params=pltpu.CompilerParams(use_tc_tiling_on_sc=False),
    out_type=jax.ShapeDtypeStruct((BATCH, EMBED), jnp.float32),
    scratch_types=[
        pltpu.VMEM((2 * CH, HALF), jnp.int32),
        pltpu.VMEM((2 * CH, HALF), jnp.int32),
        pltpu.VMEM((2 * CH, HALF, EMBED), jnp.float32),
        pltpu.VMEM((2 * CH, HALF, EMBED), jnp.float32),
        pltpu.VMEM((BPW, EMBED), jnp.float32),
        pltpu.SemaphoreType.DMA,
        pltpu.SemaphoreType.DMA,
    ],
)
def _pooled_sum(x_hbm, table_hbm, out_hbm,
                idx_a, idx_b, rows_a, rows_b, acc_v, sem_a, sem_b):
    wid = lax.axis_index("s") * NC + lax.axis_index("c")
    base = wid * BPW

    def stage(ci, idx_v, rows_v, sem):
        pltpu.sync_copy(
            x_hbm.at[pl.ds((base + ci * CH) * 2, 2 * CH)], idx_v)
        for j in range(2 * CH):
            pltpu.async_copy(table_hbm.at[idx_v.at[j]], rows_v.at[j], sem)

    def drain(rows_v, sem):
        for j in range(2 * CH):
            pltpu.make_async_copy(
                table_hbm.at[pl.ds(0, HALF)], rows_v.at[j], sem).wait()

    def consume(ci, rows_v):
        for r in range(CH):
            def sbody(i, acc, _r=r):
                a0, a1 = acc
                for k in range(UNROLL):
                    s = i * UNROLL + k
                    a0 = a0 + rows_v[2 * _r, s, pl.ds(0, L)]
                    a1 = a1 + rows_v[2 * _r, s, pl.ds(L, L)]
                    a0 = a0 + rows_v[2 * _r + 1, s, pl.ds(0, L)]
                    a1 = a1 + rows_v[2 * _r + 1, s, pl.ds(L, L)]
                return a0, a1
            z = jnp.zeros((L,), jnp.float32)
            a0, a1 = lax.fori_loop(0, HALF // UNROLL, sbody, (z, z))
            acc_v[ci * CH + r, pl.ds(0, L)] = a0
            acc_v[ci * CH + r, pl.ds(L, L)] = a1

    stage(0, idx_a, rows_a, sem_a)

    def pair_body(j, carry):
        stage(2 * j + 1, idx_b, rows_b, sem_b)
        drain(rows_a, sem_a)
        consume(2 * j, rows_a)

        @pl.when(j + 1 < NCHUNK // 2)
        def _():
            stage(2 * j + 2, idx_a, rows_a, sem_a)

        drain(rows_b, sem_b)
        consume(2 * j + 1, rows_b)
        return carry

    lax.fori_loop(0, NCHUNK // 2, pair_body, 0)
    pltpu.sync_copy(acc_v, out_hbm.at[pl.ds(base, BPW)])


def _head_body(p_ref, w_ref, b_ref, o_ref):
    pooled = p_ref[...] * (1.0 / SEQ)
    o_ref[...] = (
        jnp.dot(pooled, w_ref[...], preferred_element_type=jnp.float32,
                precision=lax.Precision.HIGHEST)
        + b_ref[...]
    )


_head = pl.pallas_call(
    _head_body,
    out_shape=jax.ShapeDtypeStruct((BATCH, NCLS), jnp.float32),
)


def kernel(x, table, W, b):
    x2 = x.astype(jnp.int32).reshape(BATCH * 2, HALF)
    pooled = _pooled_sum(x2, table)
    return _head(pooled, W, b.reshape(1, NCLS))


# speedup vs baseline: 1.2370x; 1.0127x over previous
import functools

import jax
import jax.numpy as jnp
from jax import lax
from jax.experimental import pallas as pl
from jax.experimental.pallas import tpu as pltpu
from jax.experimental.pallas import tpu_sc as plsc

VOCAB = 1000000
EMBED = 32
NCLS = 2
BATCH = 4096
SEQ = 200

NC = 2
NS = 16
L = 16
NW = NC * NS
BPW = BATCH // NW
CH = 8
NCHUNK = BPW // CH
HALF = SEQ // 2
UNROLL = 4

_mesh = plsc.VectorSubcoreMesh(core_axis_name="c", subcore_axis_name="s")


@functools.partial(
    pl.kernel,
    mesh=_mesh,
    compiler_params=pltpu.CompilerParams(use_tc_tiling_on_sc=False),
    out_type=jax.ShapeDtypeStruct((BATCH, EMBED), jnp.float32),
    scratch_types=[
        pltpu.VMEM((2 * CH, HALF), jnp.int32),
        pltpu.VMEM((2 * CH, HALF), jnp.int32),
        pltpu.VMEM((2 * CH, HALF, EMBED), jnp.float32),
        pltpu.VMEM((2 * CH, HALF, EMBED), jnp.float32),
        pltpu.VMEM((BPW, EMBED), jnp.float32),
        pltpu.SemaphoreType.DMA,
        pltpu.SemaphoreType.DMA,
    ],
)
def _pooled_sum(x_hbm, table_hbm, out_hbm,
                idx_a, idx_b, rows_a, rows_b, acc_v, sem_a, sem_b):
    wid = lax.axis_index("s") * NC + lax.axis_index("c")
    base = wid * BPW

    def stage(ci, idx_v, rows_v, sem):
        pltpu.sync_copy(
            x_hbm.at[pl.ds((base + ci * CH) * 2, 2 * CH)], idx_v)
        for j in range(2 * CH):
            pltpu.async_copy(table_hbm.at[idx_v.at[j]], rows_v.at[j], sem)

    def drain(rows_v, sem):
        for j in range(2 * CH):
            pltpu.make_async_copy(
                table_hbm.at[pl.ds(0, HALF)], rows_v.at[j], sem).wait()

    def consume(ci, rows_v):
        for r in range(CH):
            def sbody(i, acc, _r=r):
                a0, a1 = acc
                for k in range(UNROLL):
                    s = i * UNROLL + k
                    a0 = a0 + rows_v[2 * _r, s, pl.ds(0, L)]
                    a1 = a1 + rows_v[2 * _r, s, pl.ds(L, L)]
                    a0 = a0 + rows_v[2 * _r + 1, s, pl.ds(0, L)]
                    a1 = a1 + rows_v[2 * _r + 1, s, pl.ds(L, L)]
                return a0, a1
            z = jnp.zeros((L,), jnp.float32)
            a0, a1 = lax.fori_loop(0, HALF // UNROLL, sbody, (z, z))
            acc_v[ci * CH + r, pl.ds(0, L)] = a0
            acc_v[ci * CH + r, pl.ds(L, L)] = a1

    stage(0, idx_a, rows_a, sem_a)

    def pair_body(j, carry):
        stage(2 * j + 1, idx_b, rows_b, sem_b)
        drain(rows_a, sem_a)
        consume(2 * j, rows_a)

        @pl.when(j + 1 < NCHUNK // 2)
        def _():
            stage(2 * j + 2, idx_a, rows_a, sem_a)

        drain(rows_b, sem_b)
        consume(2 * j + 1, rows_b)
        return carry

    lax.fori_loop(0, NCHUNK // 2, pair_body, 0)
    pltpu.sync_copy(acc_v, out_hbm.at[pl.ds(base, BPW)])


def _head_body(p_ref, w_ref, b_ref, o_ref):
    pooled = p_ref[...] * (1.0 / SEQ)
    o_ref[...] = (
        jnp.dot(pooled, w_ref[...], preferred_element_type=jnp.float32,
                precision=lax.Precision.HIGHEST)
        + b_ref[...]
    )


_head = pl.pallas_call(
    _head_body,
    out_shape=jax.ShapeDtypeStruct((BATCH, NCLS), jnp.float32),
)


def kernel(x, table, W, b):
    x2 = x.astype(jnp.int32).reshape(BATCH * 2, HALF)
    pooled = _pooled_sum(x2, table)
    return _head(pooled, W, b.reshape(1, NCLS))
